# initial kernel scaffold (unmeasured)
import jax
import jax.numpy as jnp
from jax import lax
from jax.experimental import pallas as pl
from jax.experimental.pallas import tpu as pltpu


def kernel(
    x,
):
    def body(*refs):
        pass

    out_shape = jax.ShapeDtypeStruct(..., jnp.float32)
    return pl.pallas_call(body, out_shape=out_shape)(...)


# baseline (device time: 19441 ns/iter reference)
import jax
import jax.numpy as jnp
from jax import lax
from jax.experimental import pallas as pl
from jax.experimental.pallas import tpu as pltpu

N_DEV = 8


def kernel(x):
    m, n = x.shape

    def body(x_ref, out_ref, comm_ref, send_sems, recv_sems):
        my = lax.axis_index("i")
        left = (my - 1) % N_DEV
        right = (my + 1) % N_DEV

        barrier_sem = pltpu.get_barrier_semaphore()
        for nbr in [left, right]:
            pl.semaphore_signal(
                barrier_sem, inc=1,
                device_id=(nbr,), device_id_type=pl.DeviceIdType.MESH,
            )
        pl.semaphore_wait(barrier_sem, 2)

        lc = x_ref[:, :].astype(jnp.float32)
        s = 1
        while s < m:
            shifted = jnp.concatenate(
                [jnp.ones((s, n), jnp.float32), lc[:-s, :]], axis=0
            )
            lc = lc * shifted
            s *= 2

        comm_ref[N_DEV - 1, :, :] = lc[m - 1:m, :]

        for h in range(N_DEV - 1):
            src_slot = N_DEV - 1 if h == 0 else h - 1
            rdma = pltpu.make_async_remote_copy(
                src_ref=comm_ref.at[src_slot],
                dst_ref=comm_ref.at[h],
                send_sem=send_sems.at[h],
                recv_sem=recv_sems.at[h],
                device_id=(right,),
                device_id_type=pl.DeviceIdType.MESH,
            )
            rdma.start()
            rdma.wait()

        prefix = jnp.ones((1, n), jnp.float32)
        for h in range(N_DEV - 1):
            prefix = prefix * jnp.where(h < my, comm_ref[h, :, :], 1.0)

        out_ref[:, :] = lc * prefix

    return pl.pallas_call(
        body,
        out_shape=jax.ShapeDtypeStruct((m, n), jnp.float32),
        in_specs=[pl.BlockSpec(memory_space=pltpu.VMEM)],
        out_specs=pl.BlockSpec(memory_space=pltpu.VMEM),
        scratch_shapes=[
            pltpu.VMEM((N_DEV, 1, n), jnp.float32),
            pltpu.SemaphoreType.DMA((N_DEV - 1,)),
            pltpu.SemaphoreType.DMA((N_DEV - 1,)),
        ],
        compiler_params=pltpu.CompilerParams(collective_id=0),
    )(x)


# device time: 8596 ns/iter; 2.2616x vs baseline; 2.2616x over previous
import jax
import jax.numpy as jnp
from jax import lax
from jax.experimental import pallas as pl
from jax.experimental.pallas import tpu as pltpu

N_DEV = 8
SHIFTS = (1, 2, 4)


def kernel(x):
    m, n = x.shape

    def body(x_ref, out_ref, send_ref, recv_ref, send_sems, recv_sems):
        my = lax.axis_index("i")

        recv_ref[:, :, :] = jnp.ones((len(SHIFTS), 1, n), jnp.float32)

        barrier_sem = pltpu.get_barrier_semaphore()
        for s in SHIFTS:
            @pl.when(my - s >= 0)
            def _():
                pl.semaphore_signal(
                    barrier_sem, inc=1,
                    device_id=(my - s,), device_id_type=pl.DeviceIdType.MESH,
                )
        for s in SHIFTS:
            @pl.when(my + s < N_DEV)
            def _():
                pl.semaphore_wait(barrier_sem, 1)

        t = x_ref[:, :].astype(jnp.float32)
        h = m
        while h > 1:
            h //= 2
            t = t[:h, :] * t[h:, :]
        send_ref[0, :, :] = t

        def rdma(k, s):
            return pltpu.make_async_remote_copy(
                src_ref=send_ref.at[k],
                dst_ref=recv_ref.at[k],
                send_sem=send_sems.at[k],
                recv_sem=recv_sems.at[k],
                device_id=(jnp.minimum(my + s, N_DEV - 1),),
                device_id_type=pl.DeviceIdType.MESH,
            )

        @pl.when(my + SHIFTS[0] < N_DEV)
        def _():
            rdma(0, SHIFTS[0]).start()

        lc = x_ref[:, :].astype(jnp.float32)
        s_ = 1
        while s_ < m:
            shifted = jnp.concatenate(
                [jnp.ones((s_, n), jnp.float32), lc[:-s_, :]], axis=0
            )
            lc = lc * shifted
            s_ *= 2

        for k, s in enumerate(SHIFTS):
            @pl.when(my - s >= 0)
            def _():
                rdma(k, s).wait_recv()
            if k + 1 < len(SHIFTS):
                s_next = SHIFTS[k + 1]
                send_ref[k + 1, :, :] = send_ref[k, :, :] * recv_ref[k, :, :]
                @pl.when(my + s_next < N_DEV)
                def _():
                    rdma(k + 1, s_next).start()

        e = recv_ref[0, :, :] * recv_ref[1, :, :] * recv_ref[2, :, :]
        out_ref[:, :] = lc * e

        for k, s in enumerate(SHIFTS):
            @pl.when(my + s < N_DEV)
            def _():
                rdma(k, s).wait_send()

    return pl.pallas_call(
        body,
        out_shape=jax.ShapeDtypeStruct((m, n), jnp.float32),
        in_specs=[pl.BlockSpec(memory_space=pltpu.VMEM)],
        out_specs=pl.BlockSpec(memory_space=pltpu.VMEM),
        scratch_shapes=[
            pltpu.VMEM((len(SHIFTS), 1, n), jnp.float32),
            pltpu.VMEM((len(SHIFTS), 1, n), jnp.float32),
            pltpu.SemaphoreType.DMA((len(SHIFTS),)),
            pltpu.SemaphoreType.DMA((len(SHIFTS),)),
        ],
        compiler_params=pltpu.CompilerParams(collective_id=0),
    )(x)


# device time: 8510 ns/iter; 2.2845x vs baseline; 1.0101x over previous
import jax
import jax.numpy as jnp
from jax import lax
from jax.experimental import pallas as pl
from jax.experimental.pallas import tpu as pltpu

N_DEV = 8
SHIFTS = (1, 2, 4)


def kernel(x):
    m, n = x.shape

    def body(x_ref, out_ref, send_ref, recv_ref, send_sems, recv_sems):
        my = lax.axis_index("i")

        recv_ref[:, :, :] = jnp.ones((len(SHIFTS), 1, n), jnp.float32)

        barrier_sem = pltpu.get_barrier_semaphore()
        for s in SHIFTS:
            @pl.when(my - s >= 0)
            def _():
                pl.semaphore_signal(
                    barrier_sem, inc=1,
                    device_id=(my - s,), device_id_type=pl.DeviceIdType.MESH,
                )

        t = x_ref[:, :].astype(jnp.float32)
        h = m
        while h > 1:
            h //= 2
            t = t[:h, :] * t[h:, :]
        send_ref[0, :, :] = t

        for s in SHIFTS:
            @pl.when(my + s < N_DEV)
            def _():
                pl.semaphore_wait(barrier_sem, 1)

        def rdma(k, s):
            return pltpu.make_async_remote_copy(
                src_ref=send_ref.at[k],
                dst_ref=recv_ref.at[k],
                send_sem=send_sems.at[k],
                recv_sem=recv_sems.at[k],
                device_id=(jnp.minimum(my + s, N_DEV - 1),),
                device_id_type=pl.DeviceIdType.MESH,
            )

        @pl.when(my + SHIFTS[0] < N_DEV)
        def _():
            rdma(0, SHIFTS[0]).start()

        lc = x_ref[:, :].astype(jnp.float32)
        s_ = 1
        while s_ < m:
            shifted = jnp.concatenate(
                [jnp.ones((s_, n), jnp.float32), lc[:-s_, :]], axis=0
            )
            lc = lc * shifted
            s_ *= 2

        for k, s in enumerate(SHIFTS):
            @pl.when(my - s >= 0)
            def _():
                rdma(k, s).wait_recv()
            if k + 1 < len(SHIFTS):
                s_next = SHIFTS[k + 1]
                send_ref[k + 1, :, :] = send_ref[k, :, :] * recv_ref[k, :, :]
                @pl.when(my + s_next < N_DEV)
                def _():
                    rdma(k + 1, s_next).start()
                lc = lc * recv_ref[k, :, :]

        out_ref[:, :] = lc * recv_ref[len(SHIFTS) - 1, :, :]

        for k, s in enumerate(SHIFTS):
            @pl.when(my + s < N_DEV)
            def _():
                rdma(k, s).wait_send()

    return pl.pallas_call(
        body,
        out_shape=jax.ShapeDtypeStruct((m, n), jnp.float32),
        in_specs=[pl.BlockSpec(memory_space=pltpu.VMEM)],
        out_specs=pl.BlockSpec(memory_space=pltpu.VMEM),
        scratch_shapes=[
            pltpu.VMEM((len(SHIFTS), 1, n), jnp.float32),
            pltpu.VMEM((len(SHIFTS), 1, n), jnp.float32),
            pltpu.SemaphoreType.DMA((len(SHIFTS),)),
            pltpu.SemaphoreType.DMA((len(SHIFTS),)),
        ],
        compiler_params=pltpu.CompilerParams(collective_id=0),
    )(x)
